# Initial kernel scaffold; baseline (speedup 1.0000x reference)
import jax
import jax.numpy as jnp
from jax.experimental import pallas as pl


def kernel(positions, table):
    raise NotImplementedError("write your pallas kernel here")


# speedup vs baseline: 2.1814x; 2.1814x over previous
import functools

import jax
import jax.numpy as jnp
from jax import lax
from jax.experimental import pallas as pl
from jax.experimental.pallas import tpu as pltpu
from jax.experimental.pallas import tpu_sc as plsc

_NUM_CORES = 2
_NUM_SUBCORES = 16
_NUM_WORKERS = _NUM_CORES * _NUM_SUBCORES

_CHUNK = 64
_B_PER_W = 1024
_N_CHUNKS = _B_PER_W // _CHUNK


def _make_gather(n_rows: int, d_model: int):
  mesh = plsc.VectorSubcoreMesh(core_axis_name="c", subcore_axis_name="s")

  @functools.partial(
      pl.kernel,
      mesh=mesh,
      out_type=jax.ShapeDtypeStruct((n_rows, d_model), jnp.float32),
      scratch_types=[
          pltpu.VMEM((_N_CHUNKS, _CHUNK), jnp.int32),
          pltpu.VMEM((_CHUNK, d_model), jnp.float32),
          pltpu.SemaphoreType.DMA,
      ],
  )
  def gather_kernel(idx_hbm, table_hbm, out_hbm, idx_v, rows_v, sem):
    wid = lax.axis_index("s") * _NUM_CORES + lax.axis_index("c")
    base = wid * _B_PER_W
    pltpu.sync_copy(idx_hbm.at[wid], idx_v)

    @pl.loop(0, _N_CHUNKS)
    def _chunk(c):
      pltpu.async_copy(table_hbm.at[idx_v.at[c]], rows_v, sem).wait()
      pltpu.sync_copy(rows_v, out_hbm.at[pl.ds(base + c * _CHUNK, _CHUNK)])

  return gather_kernel


def kernel(positions, table):
  b, t = positions.shape
  n = b * t
  idx = positions.reshape(_NUM_WORKERS, _N_CHUNKS, _CHUNK).astype(jnp.int32)
  out = _make_gather(n, table.shape[1])(idx, table)
  return out.reshape(b, t, table.shape[1])


# speedup vs baseline: 2.3049x; 1.0566x over previous
import functools

import jax
import jax.numpy as jnp
from jax import lax
from jax.experimental import pallas as pl
from jax.experimental.pallas import tpu as pltpu
from jax.experimental.pallas import tpu_sc as plsc

_NUM_CORES = 2
_NUM_SUBCORES = 16
_NUM_WORKERS = _NUM_CORES * _NUM_SUBCORES

_CHUNK = 32
_NBUF = 2
_B_PER_W = 1024
_N_CHUNKS = _B_PER_W // _CHUNK


def _make_gather(n_rows: int, d_model: int):
  mesh = plsc.VectorSubcoreMesh(core_axis_name="c", subcore_axis_name="s")

  @functools.partial(
      pl.kernel,
      mesh=mesh,
      out_type=jax.ShapeDtypeStruct((n_rows, d_model), jnp.float32),
      scratch_types=[
          pltpu.VMEM((_N_CHUNKS, _CHUNK), jnp.int32),
          pltpu.VMEM((_NBUF, _CHUNK, d_model), jnp.float32),
          pltpu.SemaphoreType.DMA,
          pltpu.SemaphoreType.DMA,
          pltpu.SemaphoreType.DMA,
      ],
  )
  def gather_kernel(idx_hbm, table_hbm, out_hbm, idx_v, rows_v, gsem, wsem0,
                    wsem1):
    wid = lax.axis_index("s") * _NUM_CORES + lax.axis_index("c")
    base = wid * _B_PER_W
    pltpu.sync_copy(idx_hbm.at[wid], idx_v)
    wsems = (wsem0, wsem1)

    @pl.loop(0, _N_CHUNKS, step=_NBUF)
    def _chunk(c0):
      for b in range(_NBUF):
        c = c0 + b
        dst = out_hbm.at[pl.ds(base + c * _CHUNK, _CHUNK)]
        write = pltpu.make_async_copy(rows_v.at[b], dst, wsems[b])

        @pl.when(c0 > 0)
        def _():
          write.wait()

        pltpu.async_copy(table_hbm.at[idx_v.at[c]], rows_v.at[b], gsem).wait()
        write.start()

    for b in range(_NBUF):
      c = _N_CHUNKS - _NBUF + b
      dst = out_hbm.at[pl.ds(base + c * _CHUNK, _CHUNK)]
      pltpu.make_async_copy(rows_v.at[b], dst, wsems[b]).wait()

  return gather_kernel


def kernel(positions, table):
  b, t = positions.shape
  n = b * t
  idx = positions.reshape(_NUM_WORKERS, _N_CHUNKS, _CHUNK).astype(jnp.int32)
  out = _make_gather(n, table.shape[1])(idx, table)
  return out.reshape(b, t, table.shape[1])


# speedup vs baseline: 2.3651x; 1.0261x over previous
import functools

import jax
import jax.numpy as jnp
from jax import lax
from jax.experimental import pallas as pl
from jax.experimental.pallas import tpu as pltpu
from jax.experimental.pallas import tpu_sc as plsc

_NUM_CORES = 2
_NUM_SUBCORES = 16
_NUM_WORKERS = _NUM_CORES * _NUM_SUBCORES

_CHUNK = 16
_NBUF = 4
_B_PER_W = 1024
_N_CHUNKS = _B_PER_W // _CHUNK


def _make_gather(n_rows: int, d_model: int):
  mesh = plsc.VectorSubcoreMesh(core_axis_name="c", subcore_axis_name="s")

  @functools.partial(
      pl.kernel,
      mesh=mesh,
      out_type=jax.ShapeDtypeStruct((n_rows, d_model), jnp.float32),
      scratch_types=[
          pltpu.VMEM((_N_CHUNKS, _CHUNK), jnp.int32),
          pltpu.VMEM((_NBUF, _CHUNK, d_model), jnp.float32),
          [pltpu.SemaphoreType.DMA] * _NBUF,
          [pltpu.SemaphoreType.DMA] * _NBUF,
      ],
  )
  def gather_kernel(idx_hbm, table_hbm, out_hbm, idx_v, rows_v, gsems, wsems):
    wid = lax.axis_index("s") * _NUM_CORES + lax.axis_index("c")
    base = wid * _B_PER_W
    pltpu.sync_copy(idx_hbm.at[wid], idx_v)

    def g_copy(c, b):
      return pltpu.make_async_copy(
          table_hbm.at[idx_v.at[c]], rows_v.at[b], gsems[b])

    def w_copy(c, b):
      dst = out_hbm.at[pl.ds(base + c * _CHUNK, _CHUNK)]
      return pltpu.make_async_copy(rows_v.at[b], dst, wsems[b])

    for c in range(_NBUF - 1):
      g_copy(c, c).start()

    @pl.loop(0, _N_CHUNKS, step=_NBUF)
    def _chunks(c0):
      for b in range(_NBUF):
        c = c0 + b
        g_copy(c, b).wait()
        w_copy(c, b).start()
        b2 = (b + _NBUF - 1) % _NBUF

        @pl.when(c > 0)
        def _():
          w_copy(c - 1, b2).wait()

        @pl.when(c + _NBUF - 1 < _N_CHUNKS)
        def _():
          g_copy(c + _NBUF - 1, b2).start()

    w_copy(_N_CHUNKS - 1, (_N_CHUNKS - 1) % _NBUF).wait()

  return gather_kernel


def kernel(positions, table):
  b, t = positions.shape
  n = b * t
  idx = positions.reshape(_NUM_WORKERS, _N_CHUNKS, _CHUNK).astype(jnp.int32)
  out = _make_gather(n, table.shape[1])(idx, table)
  return out.reshape(b, t, table.shape[1])
